# baseline (speedup 1.0000x reference)
import functools

import numpy as np

import jax
import jax.numpy as jnp
from jax import lax
from jax.experimental import pallas as pl
from jax.experimental.pallas import tpu as pltpu
from jax.experimental.pallas import tpu_sc as plsc

VOCAB = 1000000
D = 16
D_ATT = 16
N_NEGS = 20
PAD_IDX = 0
B = 1024
L = 50
K = N_NEGS + 1

KP = 24
LP = 56
KG = KP // 8
BB = 64

NQ = B * KP
NP = B * LP
NW = 32
QW = NQ // NW
PW = NP // NW
CH = 128


def _gather_body(tvec8, cvec8, tidx8, cidx8, qout, pout,
                 idx8_q, idx8_c, sup0, sup1, sem0, sem1):
    wid = lax.axis_index("s") * 2 + lax.axis_index("c")
    qb = wid * QW
    cb = wid * PW
    pltpu.sync_copy(tidx8.at[pl.ds(qb, QW)], idx8_q)
    pltpu.sync_copy(cidx8.at[pl.ds(cb, PW)], idx8_c)
    sups = (sup0, sup1)
    sems = (sem0, sem1)
    jobs = [(tvec8, idx8_q, qout, qb + i * CH) for i in range(QW // CH)]
    jobs += [(cvec8, idx8_c, pout, cb + i * CH) for i in range(PW // CH)]

    def fire(j, slot):
        tab, idx8r, _, _ = jobs[j]
        base = (j * CH) if j < QW // CH else ((j - QW // CH) * CH)
        return pltpu.async_copy(
            tab.at[idx8r.at[pl.ds(base, CH)]], sups[slot], sems[slot])

    pend = fire(0, 0)
    for j in range(len(jobs)):
        nxt = fire(j + 1, (j + 1) % 2) if j + 1 < len(jobs) else None
        pend.wait()
        _, _, out, obase = jobs[j]
        pltpu.sync_copy(sups[j % 2], out.at[pl.ds(obase, CH)])
        pend = nxt


def _gather_call(tvectors, cvectors, tidx8, cidx8):
    tvec8 = tvectors.reshape(VOCAB // 8, 128)
    cvec8 = cvectors.reshape(VOCAB // 8, 128)
    mesh = plsc.VectorSubcoreMesh(core_axis_name="c", subcore_axis_name="s")
    f = functools.partial(
        pl.kernel,
        mesh=mesh,
        out_type=(
            jax.ShapeDtypeStruct((NQ, 128), jnp.float32),
            jax.ShapeDtypeStruct((NP, 128), jnp.float32),
        ),
        scratch_types=[
            pltpu.VMEM((QW,), jnp.int32),
            pltpu.VMEM((PW,), jnp.int32),
            pltpu.VMEM((CH, 128), jnp.float32),
            pltpu.VMEM((CH, 128), jnp.float32),
            pltpu.SemaphoreType.DMA,
            pltpu.SemaphoreType.DMA,
        ],
    )(_gather_body)
    return f(tvec8, cvec8, tidx8, cidx8)


def _dense_body(tm_ref, cm_ref, qs_ref, ps_ref, wd_ref, bt_ref, h8_ref,
                r_ref, g_ref, s_ref, pm_ref, out_ref):
    f32 = jnp.float32
    laneg = jax.lax.broadcasted_iota(jnp.int32, (1, 128), 1) >> 4
    cm = cm_ref[...]
    pm_e = ((cm & 7) == laneg).astype(f32)
    psel = ps_ref[...] * pm_e
    p_t = jnp.dot(psel, s_ref[...],
                  preferred_element_type=f32).reshape(BB, LP, 128)
    tm = tm_ref[...]
    qm_e = ((tm & 7) == laneg).astype(f32)
    qsel = qs_ref[...] * qm_e
    qtile = jnp.dot(qsel, s_ref[...],
                    preferred_element_type=f32)
    qs3 = qtile.reshape(BB * KG, 8, 128)
    q8 = jnp.zeros((BB * KG, 128), f32)
    for j in range(8):
        q8 = q8 + qs3[:, j, :] * (laneg == j).astype(f32)
    q8 = q8.reshape(BB, KG, 128)
    prod = q8[:, :, None, :] * p_t[:, None, :, :]
    prod2 = prod.reshape(BB * KG * LP, 128)
    hid = jnp.dot(prod2, wd_ref[...], preferred_element_type=f32)
    hid = jnp.maximum(hid + bt_ref[...], 0.0)
    sc = jnp.dot(hid, h8_ref[...], preferred_element_type=f32)
    sc = sc.reshape(BB, KG, LP, 8)
    mask = (cm == PAD_IDX).reshape(BB, 1, LP, 1)
    sc = jnp.where(mask, -1e9, sc)
    m = jnp.max(sc, axis=2, keepdims=True)
    e = jnp.exp(sc - m)
    attn = e * (1.0 / jnp.sum(e, axis=2, keepdims=True))
    a_l = jnp.dot(attn.reshape(BB * KG * LP, 8), r_ref[...],
                  preferred_element_type=f32).reshape(BB, KG, LP, 128)
    sub = jnp.sum(a_l * p_t[:, None, :, :], axis=2)
    sims = []
    for kgp in range(KG):
        qr_kgp = jnp.zeros((BB, 128), f32)
        for kg in range(KG):
            qr_kgp = qr_kgp + jnp.dot(q8[:, kg, :], pm_ref[kg, :, kgp, :],
                                      preferred_element_type=f32)
        sp = sub[:, kgp, :] * qr_kgp
        sims.append(jnp.dot(sp, g_ref[...],
                            preferred_element_type=f32)[:, None, :])
    sim = jnp.concatenate(sims, axis=1)
    kidx = (jax.lax.broadcasted_iota(jnp.int32, (KG, 8), 0) * 8
            + jax.lax.broadcasted_iota(jnp.int32, (KG, 8), 1))
    sim = jnp.where((kidx >= K)[None, :, :], -1e30, sim)
    sm = jnp.max(jnp.max(sim, axis=2, keepdims=True), axis=1, keepdims=True)
    es = jnp.exp(sim - sm)
    den = jnp.sum(jnp.sum(es, axis=2, keepdims=True), axis=1, keepdims=True)
    soft0 = es[:, 0:1, 0:1] / den + 1e-6
    part = -jnp.sum(jnp.log(soft0))
    prev = jnp.where(pl.program_id(0) == 0, 0.0, out_ref[0, 0])
    out_ref[0, 0] = prev + part


def _dense_call(tm, cm, qsuper, psuper, Wd8, bt, H8, R8, G8, S8, PM, *,
                interpret=False):
    nblk = B // BB
    return pl.pallas_call(
        _dense_body,
        grid=(nblk,),
        in_specs=[
            pl.BlockSpec((BB * KP, 1), lambda i: (i, 0)),
            pl.BlockSpec((BB * LP, 1), lambda i: (i, 0)),
            pl.BlockSpec((BB * KP, 128), lambda i: (i, 0)),
            pl.BlockSpec((BB * LP, 128), lambda i: (i, 0)),
            pl.BlockSpec((128, 128), lambda i: (0, 0)),
            pl.BlockSpec((1, 128), lambda i: (0, 0)),
            pl.BlockSpec((128, 8), lambda i: (0, 0)),
            pl.BlockSpec((8, 128), lambda i: (0, 0)),
            pl.BlockSpec((128, 8), lambda i: (0, 0)),
            pl.BlockSpec((128, 128), lambda i: (0, 0)),
            pl.BlockSpec((KG, 128, KG, 128), lambda i: (0, 0, 0, 0)),
        ],
        out_specs=pl.BlockSpec(memory_space=pltpu.SMEM),
        out_shape=jax.ShapeDtypeStruct((1, 1), jnp.float32),
        interpret=interpret,
    )(tm, cm, qsuper, psuper, Wd8, bt, H8, R8, G8, S8, PM)


def _perm_matrix():
    perm = np.zeros((KG, 128, KG, 128), np.float32)
    for kp in range(K):
        kgp, jp = divmod(kp, 8)
        for dp in range(D):
            s = dp * K + kp
            perm[s // 128, s % 128, kgp, jp * 16 + dp] = 1.0
    return jnp.asarray(perm)


def kernel(batch_titems, batch_citems, tvectors, cvectors, W_att, b_att, h_att):
    f32 = jnp.float32
    i32 = jnp.int32
    neg_key = jax.random.key(12345)
    batch_nitems = jax.random.randint(neg_key, (B, N_NEGS), 0, VOCAB)
    titems_full = jnp.concatenate(
        [batch_titems.reshape(-1, 1), batch_nitems], axis=1
    )
    tidx = jnp.pad(titems_full, ((0, 0), (0, KP - K))).reshape(NQ).astype(i32)
    cidx = jnp.pad(batch_citems, ((0, 0), (0, LP - L))).reshape(NP).astype(i32)
    qsuper, psuper = _gather_call(
        tvectors, cvectors, jnp.right_shift(tidx, 3), jnp.right_shift(cidx, 3))
    eye8 = jnp.eye(8, dtype=f32)
    Wd8 = jnp.kron(eye8, W_att.astype(f32))
    bt = jnp.tile(b_att.astype(f32), 8).reshape(1, 128)
    H8 = jnp.kron(eye8, h_att.astype(f32).reshape(D_ATT, 1))
    R8 = jnp.kron(eye8, jnp.ones((1, 16), f32))
    G8 = jnp.kron(eye8, jnp.ones((16, 1), f32))
    S8 = jnp.kron(jnp.ones((8, 8), f32), jnp.eye(D, dtype=f32))
    out = _dense_call(tidx.reshape(NQ, 1), cidx.reshape(NP, 1),
                      qsuper, psuper, Wd8, bt, H8, R8, G8, S8, _perm_matrix())
    return out[0, 0]


# speedup vs baseline: 4.7633x; 4.7633x over previous
import functools

import numpy as np

import jax
import jax.numpy as jnp
from jax import lax
from jax.experimental import pallas as pl
from jax.experimental.pallas import tpu as pltpu
from jax.experimental.pallas import tpu_sc as plsc

VOCAB = 1000000
D = 16
D_ATT = 16
N_NEGS = 20
PAD_IDX = 0
B = 1024
L = 50
K = N_NEGS + 1

KP = 24
LP = 56
KG = KP // 8
BB = 64

NQ = B * KP
NP = B * LP
NW = 32
QW = NQ // NW
PW = NP // NW
CH = 128


def _gather_body(tvec, cvec, tidx, cidx, qout, pout,
                 idx_q, idx_c, buf0, buf1, sem0, sem1):
    wid = lax.axis_index("s") * 2 + lax.axis_index("c")
    qb = wid * QW
    cb = wid * PW
    pltpu.sync_copy(tidx.at[pl.ds(qb, QW)], idx_q)
    pltpu.sync_copy(cidx.at[pl.ds(cb, PW)], idx_c)
    bufs = (buf0, buf1)
    sems = (sem0, sem1)
    jobs = [(tvec, idx_q, i * CH, qout, qb + i * CH)
            for i in range(QW // CH)]
    jobs += [(cvec, idx_c, i * CH, pout, cb + i * CH)
             for i in range(PW // CH)]

    def fire(j, slot):
        tab, idxr, base, _, _ = jobs[j]
        return pltpu.async_copy(
            tab.at[idxr.at[pl.ds(base, CH)]], bufs[slot], sems[slot])

    pend = fire(0, 0)
    for j in range(len(jobs)):
        nxt = fire(j + 1, (j + 1) % 2) if j + 1 < len(jobs) else None
        pend.wait()
        _, _, _, out, obase = jobs[j]
        pltpu.sync_copy(bufs[j % 2], out.at[pl.ds(obase, CH)])
        pend = nxt


def _gather_call(tvectors, cvectors, tidx, cidx):
    mesh = plsc.VectorSubcoreMesh(core_axis_name="c", subcore_axis_name="s")
    f = functools.partial(
        pl.kernel,
        mesh=mesh,
        out_type=(
            jax.ShapeDtypeStruct((NQ, D), jnp.float32),
            jax.ShapeDtypeStruct((NP, D), jnp.float32),
        ),
        scratch_types=[
            pltpu.VMEM((QW,), jnp.int32),
            pltpu.VMEM((PW,), jnp.int32),
            pltpu.VMEM((CH, D), jnp.float32),
            pltpu.VMEM((CH, D), jnp.float32),
            pltpu.SemaphoreType.DMA,
            pltpu.SemaphoreType.DMA,
        ],
        compiler_params=pltpu.CompilerParams(use_tc_tiling_on_sc=True),
    )(_gather_body)
    return f(tvectors, cvectors, tidx, cidx)


def _dense_body(cm_ref, q_ref, p_ref, wd_ref, bt_ref, h8_ref,
                r_ref, g_ref, t_ref, pm_ref, out_ref):
    f32 = jnp.float32
    laneg = jax.lax.broadcasted_iota(jnp.int32, (1, 128), 1) >> 4
    p_t = jnp.dot(p_ref[...], t_ref[...],
                  preferred_element_type=f32).reshape(BB, LP, 128)
    q_t = jnp.dot(q_ref[...], t_ref[...],
                  preferred_element_type=f32)
    qs3 = q_t.reshape(BB * KG, 8, 128)
    q8 = jnp.zeros((BB * KG, 128), f32)
    for j in range(8):
        q8 = q8 + qs3[:, j, :] * (laneg == j).astype(f32)
    q8 = q8.reshape(BB, KG, 128)
    prod = q8[:, :, None, :] * p_t[:, None, :, :]
    prod2 = prod.reshape(BB * KG * LP, 128)
    hid = jnp.dot(prod2, wd_ref[...], preferred_element_type=f32)
    hid = jnp.maximum(hid + bt_ref[...], 0.0)
    sc = jnp.dot(hid, h8_ref[...], preferred_element_type=f32)
    sc = sc.reshape(BB, KG, LP, 8)
    cm = cm_ref[...]
    mask = (cm == PAD_IDX).reshape(BB, 1, LP, 1)
    sc = jnp.where(mask, -1e9, sc)
    m = jnp.max(sc, axis=2, keepdims=True)
    e = jnp.exp(sc - m)
    attn = e * (1.0 / jnp.sum(e, axis=2, keepdims=True))
    a_l = jnp.dot(attn.reshape(BB * KG * LP, 8), r_ref[...],
                  preferred_element_type=f32).reshape(BB, KG, LP, 128)
    sub = jnp.sum(a_l * p_t[:, None, :, :], axis=2)
    sims = []
    for kgp in range(KG):
        qr_kgp = jnp.zeros((BB, 128), f32)
        for kg in range(KG):
            qr_kgp = qr_kgp + jnp.dot(q8[:, kg, :], pm_ref[kg, :, kgp, :],
                                      preferred_element_type=f32)
        sp = sub[:, kgp, :] * qr_kgp
        sims.append(jnp.dot(sp, g_ref[...],
                            preferred_element_type=f32)[:, None, :])
    sim = jnp.concatenate(sims, axis=1)
    kidx = (jax.lax.broadcasted_iota(jnp.int32, (KG, 8), 0) * 8
            + jax.lax.broadcasted_iota(jnp.int32, (KG, 8), 1))
    sim = jnp.where((kidx >= K)[None, :, :], -1e30, sim)
    sm = jnp.max(jnp.max(sim, axis=2, keepdims=True), axis=1, keepdims=True)
    es = jnp.exp(sim - sm)
    den = jnp.sum(jnp.sum(es, axis=2, keepdims=True), axis=1, keepdims=True)
    soft0 = es[:, 0:1, 0:1] / den + 1e-6
    part = -jnp.sum(jnp.log(soft0))
    prev = jnp.where(pl.program_id(0) == 0, 0.0, out_ref[0, 0])
    out_ref[0, 0] = prev + part


def _dense_call(cm, qrows, prows, Wd8, bt, H8, R8, G8, T16, PM, *,
                interpret=False):
    nblk = B // BB
    return pl.pallas_call(
        _dense_body,
        grid=(nblk,),
        in_specs=[
            pl.BlockSpec((BB * LP, 1), lambda i: (i, 0)),
            pl.BlockSpec((BB * KP, D), lambda i: (i, 0)),
            pl.BlockSpec((BB * LP, D), lambda i: (i, 0)),
            pl.BlockSpec((128, 128), lambda i: (0, 0)),
            pl.BlockSpec((1, 128), lambda i: (0, 0)),
            pl.BlockSpec((128, 8), lambda i: (0, 0)),
            pl.BlockSpec((8, 128), lambda i: (0, 0)),
            pl.BlockSpec((128, 8), lambda i: (0, 0)),
            pl.BlockSpec((D, 128), lambda i: (0, 0)),
            pl.BlockSpec((KG, 128, KG, 128), lambda i: (0, 0, 0, 0)),
        ],
        out_specs=pl.BlockSpec(memory_space=pltpu.SMEM),
        out_shape=jax.ShapeDtypeStruct((1, 1), jnp.float32),
        interpret=interpret,
    )(cm, qrows, prows, Wd8, bt, H8, R8, G8, T16, PM)


def _perm_matrix():
    perm = np.zeros((KG, 128, KG, 128), np.float32)
    for kp in range(K):
        kgp, jp = divmod(kp, 8)
        for dp in range(D):
            s = dp * K + kp
            perm[s // 128, s % 128, kgp, jp * 16 + dp] = 1.0
    return jnp.asarray(perm)


def kernel(batch_titems, batch_citems, tvectors, cvectors, W_att, b_att, h_att):
    f32 = jnp.float32
    i32 = jnp.int32
    neg_key = jax.random.key(12345)
    batch_nitems = jax.random.randint(neg_key, (B, N_NEGS), 0, VOCAB)
    titems_full = jnp.concatenate(
        [batch_titems.reshape(-1, 1), batch_nitems], axis=1
    )
    tidx = jnp.pad(titems_full, ((0, 0), (0, KP - K))).reshape(NQ).astype(i32)
    cidx = jnp.pad(batch_citems, ((0, 0), (0, LP - L))).reshape(NP).astype(i32)
    qrows = jnp.take(tvectors, tidx, axis=0)
    prows = jnp.take(cvectors, cidx, axis=0)
    eye8 = jnp.eye(8, dtype=f32)
    Wd8 = jnp.kron(eye8, W_att.astype(f32))
    bt = jnp.tile(b_att.astype(f32), 8).reshape(1, 128)
    H8 = jnp.kron(eye8, h_att.astype(f32).reshape(D_ATT, 1))
    R8 = jnp.kron(eye8, jnp.ones((1, 16), f32))
    G8 = jnp.kron(eye8, jnp.ones((16, 1), f32))
    T16 = jnp.kron(jnp.ones((1, 8), f32), jnp.eye(D, dtype=f32))
    out = _dense_call(cidx.reshape(NP, 1), qrows, prows,
                      Wd8, bt, H8, R8, G8, T16, _perm_matrix())
    return out[0, 0]


# speedup vs baseline: 5.3639x; 1.1261x over previous
import functools

import numpy as np

import jax
import jax.numpy as jnp
from jax import lax
from jax.experimental import pallas as pl
from jax.experimental.pallas import tpu as pltpu
from jax.experimental.pallas import tpu_sc as plsc

VOCAB = 1000000
D = 16
D_ATT = 16
N_NEGS = 20
PAD_IDX = 0
B = 1024
L = 50
K = N_NEGS + 1

KP = 24
LP = 56
KG = KP // 8
BB = 64

NQ = B * KP
NP = B * LP
NW = 32
QW = NQ // NW
PW = NP // NW
CH = 128


def _gather_body(tvec, cvec, tidx, cidx, qout, pout,
                 idx_q, idx_c, buf0, buf1, sem0, sem1):
    wid = lax.axis_index("s") * 2 + lax.axis_index("c")
    qb = wid * QW
    cb = wid * PW
    pltpu.sync_copy(tidx.at[pl.ds(qb, QW)], idx_q)
    pltpu.sync_copy(cidx.at[pl.ds(cb, PW)], idx_c)
    bufs = (buf0, buf1)
    sems = (sem0, sem1)
    jobs = [(tvec, idx_q, i * CH, qout, qb + i * CH)
            for i in range(QW // CH)]
    jobs += [(cvec, idx_c, i * CH, pout, cb + i * CH)
             for i in range(PW // CH)]

    def fire(j, slot):
        tab, idxr, base, _, _ = jobs[j]
        return pltpu.async_copy(
            tab.at[idxr.at[pl.ds(base, CH)]], bufs[slot], sems[slot])

    pend = fire(0, 0)
    for j in range(len(jobs)):
        nxt = fire(j + 1, (j + 1) % 2) if j + 1 < len(jobs) else None
        pend.wait()
        _, _, _, out, obase = jobs[j]
        pltpu.sync_copy(bufs[j % 2], out.at[pl.ds(obase, CH)])
        pend = nxt


def _gather_call(tvectors, cvectors, tidx, cidx):
    mesh = plsc.VectorSubcoreMesh(core_axis_name="c", subcore_axis_name="s")
    f = functools.partial(
        pl.kernel,
        mesh=mesh,
        out_type=(
            jax.ShapeDtypeStruct((NQ, D), jnp.float32),
            jax.ShapeDtypeStruct((NP, D), jnp.float32),
        ),
        scratch_types=[
            pltpu.VMEM((QW,), jnp.int32),
            pltpu.VMEM((PW,), jnp.int32),
            pltpu.VMEM((CH, D), jnp.float32),
            pltpu.VMEM((CH, D), jnp.float32),
            pltpu.SemaphoreType.DMA,
            pltpu.SemaphoreType.DMA,
        ],
        compiler_params=pltpu.CompilerParams(use_tc_tiling_on_sc=True),
    )(_gather_body)
    return f(tvectors, cvectors, tidx, cidx)


def _dense_body(cm_ref, q_ref, p_ref, wd_ref, bt_ref, h8_ref,
                r_ref, g_ref, s_ref, pm_ref, out_ref):
    f32 = jnp.float32
    laneg = jax.lax.broadcasted_iota(jnp.int32, (8, 128), 1) >> 4
    gmask = (laneg == jax.lax.broadcasted_iota(jnp.int32, (8, 128), 0))
    pexp = jnp.broadcast_to(p_ref[...][:, None, :], (BB * LP // 8, 8, 128))
    psel = pexp * gmask.astype(f32)[None, :, :]
    p_t = jnp.dot(psel.reshape(BB * LP, 128), s_ref[...],
                  preferred_element_type=f32).reshape(BB, LP, 128)
    q8 = q_ref[...].reshape(BB, KG, 128)
    prod = q8[:, :, None, :] * p_t[:, None, :, :]
    prod2 = prod.reshape(BB * KG * LP, 128)
    hid = jnp.dot(prod2, wd_ref[...], preferred_element_type=f32)
    hid = jnp.maximum(hid + bt_ref[...], 0.0)
    sc = jnp.dot(hid, h8_ref[...], preferred_element_type=f32)
    sc = sc.reshape(BB, KG, LP, 8)
    cm = cm_ref[...]
    mask = (cm == PAD_IDX).reshape(BB, 1, LP, 1)
    sc = jnp.where(mask, -1e9, sc)
    m = jnp.max(sc, axis=2, keepdims=True)
    e = jnp.exp(sc - m)
    attn = e * (1.0 / jnp.sum(e, axis=2, keepdims=True))
    a_l = jnp.dot(attn.reshape(BB * KG * LP, 8), r_ref[...],
                  preferred_element_type=f32).reshape(BB, KG, LP, 128)
    sub = jnp.sum(a_l * p_t[:, None, :, :], axis=2)
    sims = []
    for kgp in range(KG):
        qr_kgp = jnp.zeros((BB, 128), f32)
        for kg in range(KG):
            qr_kgp = qr_kgp + jnp.dot(q8[:, kg, :], pm_ref[kg, :, kgp, :],
                                      preferred_element_type=f32)
        sp = sub[:, kgp, :] * qr_kgp
        sims.append(jnp.dot(sp, g_ref[...],
                            preferred_element_type=f32)[:, None, :])
    sim = jnp.concatenate(sims, axis=1)
    kidx = (jax.lax.broadcasted_iota(jnp.int32, (KG, 8), 0) * 8
            + jax.lax.broadcasted_iota(jnp.int32, (KG, 8), 1))
    sim = jnp.where((kidx >= K)[None, :, :], -1e30, sim)
    sm = jnp.max(jnp.max(sim, axis=2, keepdims=True), axis=1, keepdims=True)
    es = jnp.exp(sim - sm)
    den = jnp.sum(jnp.sum(es, axis=2, keepdims=True), axis=1, keepdims=True)
    soft0 = es[:, 0:1, 0:1] / den + 1e-6
    part = -jnp.sum(jnp.log(soft0))
    prev = jnp.where(pl.program_id(0) == 0, 0.0, out_ref[0, 0])
    out_ref[0, 0] = prev + part


def _dense_call(cm, q128, p128, Wd8, bt, H8, R8, G8, S8, PM, *,
                interpret=False):
    nblk = B // BB
    return pl.pallas_call(
        _dense_body,
        grid=(nblk,),
        in_specs=[
            pl.BlockSpec((BB * LP, 1), lambda i: (i, 0)),
            pl.BlockSpec((BB * KG, 128), lambda i: (i, 0)),
            pl.BlockSpec((BB * LP // 8, 128), lambda i: (i, 0)),
            pl.BlockSpec((128, 128), lambda i: (0, 0)),
            pl.BlockSpec((1, 128), lambda i: (0, 0)),
            pl.BlockSpec((128, 8), lambda i: (0, 0)),
            pl.BlockSpec((8, 128), lambda i: (0, 0)),
            pl.BlockSpec((128, 8), lambda i: (0, 0)),
            pl.BlockSpec((128, 128), lambda i: (0, 0)),
            pl.BlockSpec((KG, 128, KG, 128), lambda i: (0, 0, 0, 0)),
        ],
        out_specs=pl.BlockSpec(memory_space=pltpu.SMEM),
        out_shape=jax.ShapeDtypeStruct((1, 1), jnp.float32),
        interpret=interpret,
    )(cm, q128, p128, Wd8, bt, H8, R8, G8, S8, PM)


def _perm_matrix():
    perm = np.zeros((KG, 128, KG, 128), np.float32)
    for kp in range(K):
        kgp, jp = divmod(kp, 8)
        for dp in range(D):
            s = dp * K + kp
            perm[s // 128, s % 128, kgp, jp * 16 + dp] = 1.0
    return jnp.asarray(perm)


def kernel(batch_titems, batch_citems, tvectors, cvectors, W_att, b_att, h_att):
    f32 = jnp.float32
    i32 = jnp.int32
    neg_key = jax.random.key(12345)
    batch_nitems = jax.random.randint(neg_key, (B, N_NEGS), 0, VOCAB)
    titems_full = jnp.concatenate(
        [batch_titems.reshape(-1, 1), batch_nitems], axis=1
    )
    tidx = jnp.pad(titems_full, ((0, 0), (0, KP - K))).reshape(NQ).astype(i32)
    cidx = jnp.pad(batch_citems, ((0, 0), (0, LP - L))).reshape(NP).astype(i32)
    q128 = jnp.take(tvectors, tidx, axis=0).reshape(NQ // 8, 128)
    p128 = jnp.take(cvectors, cidx, axis=0).reshape(NP // 8, 128)
    eye8 = jnp.eye(8, dtype=f32)
    Wd8 = jnp.kron(eye8, W_att.astype(f32))
    bt = jnp.tile(b_att.astype(f32), 8).reshape(1, 128)
    H8 = jnp.kron(eye8, h_att.astype(f32).reshape(D_ATT, 1))
    R8 = jnp.kron(eye8, jnp.ones((1, 16), f32))
    G8 = jnp.kron(eye8, jnp.ones((16, 1), f32))
    S8 = jnp.kron(jnp.ones((8, 8), f32), jnp.eye(D, dtype=f32))
    out = _dense_call(cidx.reshape(NP, 1), q128, p128,
                      Wd8, bt, H8, R8, G8, S8, _perm_matrix())
    return out[0, 0]


# speedup vs baseline: 5.4926x; 1.0240x over previous
import functools

import numpy as np

import jax
import jax.numpy as jnp
from jax import lax
from jax.experimental import pallas as pl
from jax.experimental.pallas import tpu as pltpu
from jax.experimental.pallas import tpu_sc as plsc

VOCAB = 1000000
D = 16
D_ATT = 16
N_NEGS = 20
PAD_IDX = 0
B = 1024
L = 50
K = N_NEGS + 1

KP = 24
LP = 56
KG = KP // 8
BB = 128

NQ = B * KP
NP = B * LP
NW = 32
QW = NQ // NW
PW = NP // NW
CH = 128


def _gather_body(tvec, cvec, tidx, cidx, qout, pout,
                 idx_q, idx_c, buf0, buf1, sem0, sem1):
    wid = lax.axis_index("s") * 2 + lax.axis_index("c")
    qb = wid * QW
    cb = wid * PW
    pltpu.sync_copy(tidx.at[pl.ds(qb, QW)], idx_q)
    pltpu.sync_copy(cidx.at[pl.ds(cb, PW)], idx_c)
    bufs = (buf0, buf1)
    sems = (sem0, sem1)
    jobs = [(tvec, idx_q, i * CH, qout, qb + i * CH)
            for i in range(QW // CH)]
    jobs += [(cvec, idx_c, i * CH, pout, cb + i * CH)
             for i in range(PW // CH)]

    def fire(j, slot):
        tab, idxr, base, _, _ = jobs[j]
        return pltpu.async_copy(
            tab.at[idxr.at[pl.ds(base, CH)]], bufs[slot], sems[slot])

    pend = fire(0, 0)
    for j in range(len(jobs)):
        nxt = fire(j + 1, (j + 1) % 2) if j + 1 < len(jobs) else None
        pend.wait()
        _, _, _, out, obase = jobs[j]
        pltpu.sync_copy(bufs[j % 2], out.at[pl.ds(obase, CH)])
        pend = nxt


def _gather_call(tvectors, cvectors, tidx, cidx):
    mesh = plsc.VectorSubcoreMesh(core_axis_name="c", subcore_axis_name="s")
    f = functools.partial(
        pl.kernel,
        mesh=mesh,
        out_type=(
            jax.ShapeDtypeStruct((NQ, D), jnp.float32),
            jax.ShapeDtypeStruct((NP, D), jnp.float32),
        ),
        scratch_types=[
            pltpu.VMEM((QW,), jnp.int32),
            pltpu.VMEM((PW,), jnp.int32),
            pltpu.VMEM((CH, D), jnp.float32),
            pltpu.VMEM((CH, D), jnp.float32),
            pltpu.SemaphoreType.DMA,
            pltpu.SemaphoreType.DMA,
        ],
        compiler_params=pltpu.CompilerParams(use_tc_tiling_on_sc=True),
    )(_gather_body)
    return f(tvectors, cvectors, tidx, cidx)


def _dense_body(cm_ref, q_ref, p_ref, wd_ref, bt_ref, h8_ref,
                r_ref, g_ref, s_ref, pm_ref, out_ref):
    f32 = jnp.float32
    laneg = jax.lax.broadcasted_iota(jnp.int32, (8, 128), 1) >> 4
    gmask = (laneg == jax.lax.broadcasted_iota(jnp.int32, (8, 128), 0))
    pexp = jnp.broadcast_to(p_ref[...][:, None, :], (BB * LP // 8, 8, 128))
    psel = pexp * gmask.astype(f32)[None, :, :]
    p_t = jnp.dot(psel.reshape(BB * LP, 128), s_ref[...],
                  preferred_element_type=f32).reshape(BB, LP, 128)
    q8 = q_ref[...].reshape(BB, KG, 128)
    prod = q8[:, :, None, :] * p_t[:, None, :, :]
    prod2 = prod.reshape(BB * KG * LP, 128)
    hid = jnp.dot(prod2, wd_ref[...], preferred_element_type=f32)
    hid = jnp.maximum(hid + bt_ref[...], 0.0)
    sc = jnp.dot(hid, h8_ref[...], preferred_element_type=f32)
    sc = sc.reshape(BB, KG, LP, 8)
    cm = cm_ref[...]
    mask = (cm == PAD_IDX).reshape(BB, 1, LP, 1)
    sc = jnp.where(mask, -1e9, sc)
    m = jnp.max(sc, axis=2, keepdims=True)
    e = jnp.exp(sc - m)
    attn = e * (1.0 / jnp.sum(e, axis=2, keepdims=True))
    a_l = jnp.dot(attn.reshape(BB * KG * LP, 8), r_ref[...],
                  preferred_element_type=f32).reshape(BB, KG, LP, 128)
    sub = jnp.sum(a_l * p_t[:, None, :, :], axis=2)
    sims = []
    for kgp in range(KG):
        qr_kgp = jnp.zeros((BB, 128), f32)
        for kg in range(KG):
            qr_kgp = qr_kgp + jnp.dot(q8[:, kg, :], pm_ref[kg, :, kgp, :],
                                      preferred_element_type=f32)
        sp = sub[:, kgp, :] * qr_kgp
        sims.append(jnp.dot(sp, g_ref[...],
                            preferred_element_type=f32)[:, None, :])
    sim = jnp.concatenate(sims, axis=1)
    kidx = (jax.lax.broadcasted_iota(jnp.int32, (KG, 8), 0) * 8
            + jax.lax.broadcasted_iota(jnp.int32, (KG, 8), 1))
    sim = jnp.where((kidx >= K)[None, :, :], -1e30, sim)
    sm = jnp.max(jnp.max(sim, axis=2, keepdims=True), axis=1, keepdims=True)
    es = jnp.exp(sim - sm)
    den = jnp.sum(jnp.sum(es, axis=2, keepdims=True), axis=1, keepdims=True)
    soft0 = es[:, 0:1, 0:1] / den + 1e-6
    part = -jnp.sum(jnp.log(soft0))
    prev = jnp.where(pl.program_id(0) == 0, 0.0, out_ref[0, 0])
    out_ref[0, 0] = prev + part


def _dense_call(cm, q128, p128, Wd8, bt, H8, R8, G8, S8, PM, *,
                interpret=False):
    nblk = B // BB
    return pl.pallas_call(
        _dense_body,
        grid=(nblk,),
        in_specs=[
            pl.BlockSpec((BB * LP, 1), lambda i: (i, 0)),
            pl.BlockSpec((BB * KG, 128), lambda i: (i, 0)),
            pl.BlockSpec((BB * LP // 8, 128), lambda i: (i, 0)),
            pl.BlockSpec((128, 128), lambda i: (0, 0)),
            pl.BlockSpec((1, 128), lambda i: (0, 0)),
            pl.BlockSpec((128, 8), lambda i: (0, 0)),
            pl.BlockSpec((8, 128), lambda i: (0, 0)),
            pl.BlockSpec((128, 8), lambda i: (0, 0)),
            pl.BlockSpec((128, 128), lambda i: (0, 0)),
            pl.BlockSpec((KG, 128, KG, 128), lambda i: (0, 0, 0, 0)),
        ],
        out_specs=pl.BlockSpec(memory_space=pltpu.SMEM),
        out_shape=jax.ShapeDtypeStruct((1, 1), jnp.float32),
        interpret=interpret,
    )(cm, q128, p128, Wd8, bt, H8, R8, G8, S8, PM)


def _perm_matrix():
    perm = np.zeros((KG, 128, KG, 128), np.float32)
    for kp in range(K):
        kgp, jp = divmod(kp, 8)
        for dp in range(D):
            s = dp * K + kp
            perm[s // 128, s % 128, kgp, jp * 16 + dp] = 1.0
    return jnp.asarray(perm)


def kernel(batch_titems, batch_citems, tvectors, cvectors, W_att, b_att, h_att):
    f32 = jnp.float32
    i32 = jnp.int32
    neg_key = jax.random.key(12345)
    batch_nitems = jax.random.randint(neg_key, (B, N_NEGS), 0, VOCAB)
    titems_full = jnp.concatenate(
        [batch_titems.reshape(-1, 1), batch_nitems], axis=1
    )
    tidx = jnp.pad(titems_full, ((0, 0), (0, KP - K))).reshape(NQ).astype(i32)
    cidx = jnp.pad(batch_citems, ((0, 0), (0, LP - L))).reshape(NP).astype(i32)
    q128 = jnp.take(tvectors, tidx, axis=0).reshape(NQ // 8, 128)
    p128 = jnp.take(cvectors, cidx, axis=0).reshape(NP // 8, 128)
    eye8 = jnp.eye(8, dtype=f32)
    Wd8 = jnp.kron(eye8, W_att.astype(f32))
    bt = jnp.tile(b_att.astype(f32), 8).reshape(1, 128)
    H8 = jnp.kron(eye8, h_att.astype(f32).reshape(D_ATT, 1))
    R8 = jnp.kron(eye8, jnp.ones((1, 16), f32))
    G8 = jnp.kron(eye8, jnp.ones((16, 1), f32))
    S8 = jnp.kron(jnp.ones((8, 8), f32), jnp.eye(D, dtype=f32))
    out = _dense_call(cidx.reshape(NP, 1), q128, p128,
                      Wd8, bt, H8, R8, G8, S8, _perm_matrix())
    return out[0, 0]
